# baseline (speedup 1.0000x reference)
import functools

import jax
import jax.numpy as jnp
from jax import lax
from jax.experimental import pallas as pl
from jax.experimental.pallas import tpu as pltpu
from jax.experimental.pallas import tpu_sc as plsc

_CH = 32


def kernel(inputs, P):
    n_batch, s = inputs.shape
    d = P.shape[1]
    mesh = plsc.VectorSubcoreMesh(core_axis_name="c", subcore_axis_name="s")
    n_workers = mesh.num_cores * mesh.num_subcores
    rows_per_w = s // n_workers

    @functools.partial(
        pl.kernel,
        out_type=jax.ShapeDtypeStruct((n_batch, s, d), P.dtype),
        mesh=mesh,
        scratch_types=[
            pltpu.VMEM((_CH, d), P.dtype),
            pltpu.SemaphoreType.DMA,
        ],
    )
    def sc_broadcast(p_hbm, o_hbm, buf, sem):
        wid = lax.axis_index("c") * mesh.num_subcores + lax.axis_index("s")
        base = wid * rows_per_w

        @pl.loop(0, rows_per_w // _CH)
        def _(ci):
            off = base + ci * _CH
            pltpu.sync_copy(p_hbm.at[pl.ds(off, _CH)], buf)
            copies = [
                pltpu.async_copy(buf, o_hbm.at[b, pl.ds(off, _CH)], sem)
                for b in range(n_batch)
            ]
            for cp in copies:
                cp.wait()

    return sc_broadcast(P)


# speedup vs baseline: 1.0461x; 1.0461x over previous
import functools

import jax
import jax.numpy as jnp
from jax import lax
from jax.experimental import pallas as pl
from jax.experimental.pallas import tpu as pltpu
from jax.experimental.pallas import tpu_sc as plsc

_CH = 32


def kernel(inputs, P):
    n_batch, s = inputs.shape
    d = P.shape[1]
    mesh = plsc.VectorSubcoreMesh(core_axis_name="c", subcore_axis_name="s")
    n_workers = mesh.num_cores * mesh.num_subcores
    rows_per_w = s // n_workers
    n_ch = rows_per_w // _CH

    @functools.partial(
        pl.kernel,
        out_type=jax.ShapeDtypeStruct((n_batch, s, d), P.dtype),
        mesh=mesh,
        scratch_types=[
            pltpu.VMEM((2, _CH, d), P.dtype),
            pltpu.SemaphoreType.DMA((2,)),
            pltpu.SemaphoreType.DMA((2,)),
        ],
    )
    def sc_broadcast(p_hbm, o_hbm, buf, rsems, wsems):
        wid = lax.axis_index("c") * mesh.num_subcores + lax.axis_index("s")
        base = wid * rows_per_w

        def rd(ci):
            return pltpu.make_async_copy(
                p_hbm.at[pl.ds(base + ci * _CH, _CH)], buf.at[ci % 2],
                rsems.at[ci % 2])

        def wr(ci, b):
            return pltpu.make_async_copy(
                buf.at[ci % 2], o_hbm.at[b, pl.ds(base + ci * _CH, _CH)],
                wsems.at[ci % 2])

        rd(0).start()
        rd(1).start()
        for ci in range(n_ch):
            rd(ci).wait()
            for b in range(n_batch):
                wr(ci, b).start()
            if ci + 2 < n_ch:
                for b in range(n_batch):
                    wr(ci, b).wait()
                rd(ci + 2).start()
        for ci in range(max(0, n_ch - 2), n_ch):
            for b in range(n_batch):
                wr(ci, b).wait()

    return sc_broadcast(P)


# speedup vs baseline: 1.8923x; 1.8090x over previous
import jax
import jax.numpy as jnp
from jax.experimental import pallas as pl
from jax.experimental.pallas import tpu as pltpu

_NCH = 16


def _dma_body(p_hbm, o_hbm, vbuf, in_sems, out_sems):
    n_batch = o_hbm.shape[0]
    ch_rows = p_hbm.shape[0] // _NCH

    def in_copy(ch):
        return pltpu.make_async_copy(
            p_hbm.at[pl.ds(ch * ch_rows, ch_rows)],
            vbuf.at[pl.ds(ch * ch_rows, ch_rows)],
            in_sems.at[ch])

    def out_copy(ch, b):
        return pltpu.make_async_copy(
            vbuf.at[pl.ds(ch * ch_rows, ch_rows)],
            o_hbm.at[b, pl.ds(ch * ch_rows, ch_rows)],
            out_sems.at[ch])

    for ch in range(_NCH):
        in_copy(ch).start(priority=ch % 2)
    for ch in range(_NCH):
        in_copy(ch).wait()
        for b in range(n_batch):
            out_copy(ch, b).start(priority=b % 2)
    for ch in range(_NCH):
        for b in range(n_batch):
            out_copy(ch, b).wait()


def kernel(inputs, P):
    b, s = inputs.shape
    d = P.shape[1]
    return pl.pallas_call(
        _dma_body,
        in_specs=[pl.BlockSpec(memory_space=pltpu.MemorySpace.HBM)],
        out_specs=pl.BlockSpec(memory_space=pltpu.MemorySpace.HBM),
        out_shape=jax.ShapeDtypeStruct((b, s, d), P.dtype),
        scratch_shapes=[
            pltpu.VMEM((s, d), P.dtype),
            pltpu.SemaphoreType.DMA((_NCH,)),
            pltpu.SemaphoreType.DMA((_NCH,)),
        ],
    )(P)


# speedup vs baseline: 1.9357x; 1.0229x over previous
import jax
import jax.numpy as jnp
from jax.experimental import pallas as pl
from jax.experimental.pallas import tpu as pltpu

_CHUNKS = (128, 256, 512, 512, 512, 512, 512, 512, 384, 256)
_OFFS = tuple(sum(_CHUNKS[:i]) for i in range(len(_CHUNKS)))


def _dma_body(p_hbm, o_hbm, vbuf, in_sems, out_sems):
    n_batch = o_hbm.shape[0]

    def in_copy(ch):
        return pltpu.make_async_copy(
            p_hbm.at[pl.ds(_OFFS[ch], _CHUNKS[ch])],
            vbuf.at[pl.ds(_OFFS[ch], _CHUNKS[ch])],
            in_sems.at[ch])

    def out_copy(ch, b):
        return pltpu.make_async_copy(
            vbuf.at[pl.ds(_OFFS[ch], _CHUNKS[ch])],
            o_hbm.at[b, pl.ds(_OFFS[ch], _CHUNKS[ch])],
            out_sems.at[ch])

    for ch in range(len(_CHUNKS)):
        in_copy(ch).start()
    for ch in range(len(_CHUNKS)):
        in_copy(ch).wait()
        for b in range(n_batch):
            out_copy(ch, b).start(priority=b % 2)
    for ch in range(len(_CHUNKS)):
        for b in range(n_batch):
            out_copy(ch, b).wait()


def kernel(inputs, P):
    b, s = inputs.shape
    d = P.shape[1]
    return pl.pallas_call(
        _dma_body,
        in_specs=[pl.BlockSpec(memory_space=pltpu.MemorySpace.HBM)],
        out_specs=pl.BlockSpec(memory_space=pltpu.MemorySpace.HBM),
        out_shape=jax.ShapeDtypeStruct((b, s, d), P.dtype),
        scratch_shapes=[
            pltpu.VMEM((s, d), P.dtype),
            pltpu.SemaphoreType.DMA((len(_CHUNKS),)),
            pltpu.SemaphoreType.DMA((len(_CHUNKS),)),
        ],
    )(P)


# speedup vs baseline: 1.9386x; 1.0015x over previous
import jax
import jax.numpy as jnp
from jax.experimental import pallas as pl
from jax.experimental.pallas import tpu as pltpu

_NCH = 8


def _dma_body(p_hbm, o_hbm, vbuf, in_sems, out_sems):
    n_batch = o_hbm.shape[0]
    ch_rows = p_hbm.shape[0] // _NCH

    def in_copy(ch):
        return pltpu.make_async_copy(
            p_hbm.at[pl.ds(ch * ch_rows, ch_rows)],
            vbuf.at[pl.ds(ch * ch_rows, ch_rows)],
            in_sems.at[ch])

    def out_copy(ch, b):
        return pltpu.make_async_copy(
            vbuf.at[pl.ds(ch * ch_rows, ch_rows)],
            o_hbm.at[b, pl.ds(ch * ch_rows, ch_rows)],
            out_sems.at[ch])

    for ch in range(_NCH):
        in_copy(ch).start()
    for ch in range(_NCH):
        in_copy(ch).wait()
        for b in range(n_batch):
            out_copy(ch, b).start(priority=b % 2)
    for ch in range(_NCH):
        for b in range(n_batch):
            out_copy(ch, b).wait()


def kernel(inputs, P):
    b, s = inputs.shape
    d = P.shape[1]
    return pl.pallas_call(
        _dma_body,
        in_specs=[pl.BlockSpec(memory_space=pltpu.MemorySpace.HBM)],
        out_specs=pl.BlockSpec(memory_space=pltpu.MemorySpace.HBM),
        out_shape=jax.ShapeDtypeStruct((b, s, d), P.dtype),
        scratch_shapes=[
            pltpu.VMEM((s, d), P.dtype),
            pltpu.SemaphoreType.DMA((_NCH,)),
            pltpu.SemaphoreType.DMA((_NCH,)),
        ],
    )(P)
